# Initial kernel scaffold; baseline (speedup 1.0000x reference)
import jax
import jax.numpy as jnp
from jax.experimental import pallas as pl


def kernel(x, gate_weight, adaptive_bias):
    raise NotImplementedError("write your pallas kernel here")


# speedup vs baseline: 1.2068x; 1.2068x over previous
import functools

import jax
import jax.numpy as jnp
from jax import lax
from jax.experimental import pallas as pl
from jax.experimental.pallas import tpu as pltpu

DIM = 768
N_EXPERTS = 64
TOP_K = 8
BLOCK_T = 2048


def _gate_block(x_ref, w_ref, b_ref, wout_ref, iout_ref):
    x = x_ref[...]
    w = w_ref[...]
    scores = jnp.dot(x, w, preferred_element_type=jnp.float32) + b_ref[...]
    probs = jax.nn.softmax(scores, axis=-1)
    eiota = lax.broadcasted_iota(jnp.int32, probs.shape, 1)
    vals = []
    idxs = []
    total = None
    for _ in range(TOP_K):
        m = jnp.max(probs, axis=-1, keepdims=True)
        idx = jnp.min(jnp.where(probs == m, eiota, N_EXPERTS), axis=-1,
                      keepdims=True)
        probs = jnp.where(eiota == idx, -1.0, probs)
        vals.append(m)
        idxs.append(idx)
        total = m if total is None else total + m
    wts = jnp.concatenate(vals, axis=-1)
    wout_ref[...] = wts / (total + 1e-8)
    iout_ref[...] = jnp.concatenate(idxs, axis=-1)


@functools.partial(jax.jit, static_argnames=())
def kernel(x, gate_weight, adaptive_bias):
    orig_shape = x.shape
    xf = x.reshape(-1, orig_shape[-1])
    t = xf.shape[0]
    bt = min(BLOCK_T, t)
    wt = gate_weight.T
    bias = adaptive_bias.reshape(1, N_EXPERTS)
    grid = (pl.cdiv(t, bt),)
    wts, idx = pl.pallas_call(
        _gate_block,
        grid=grid,
        in_specs=[
            pl.BlockSpec((bt, DIM), lambda i: (i, 0)),
            pl.BlockSpec((DIM, N_EXPERTS), lambda i: (0, 0)),
            pl.BlockSpec((1, N_EXPERTS), lambda i: (0, 0)),
        ],
        out_specs=[
            pl.BlockSpec((bt, TOP_K), lambda i: (i, 0)),
            pl.BlockSpec((bt, TOP_K), lambda i: (i, 0)),
        ],
        out_shape=[
            jax.ShapeDtypeStruct((t, TOP_K), jnp.float32),
            jax.ShapeDtypeStruct((t, TOP_K), jnp.int32),
        ],
    )(xf, wt, bias)
    if len(orig_shape) == 3:
        wts = wts.reshape(orig_shape[0], orig_shape[1], TOP_K)
        idx = idx.reshape(orig_shape[0], orig_shape[1], TOP_K)
    return (wts, idx)


# speedup vs baseline: 1.5375x; 1.2741x over previous
import functools

import jax
import jax.numpy as jnp
from jax import lax
from jax.experimental import pallas as pl

DIM = 768
N_EXPERTS = 64
TOP_K = 8
BLOCK_T = 2048

_IDX_MASK = N_EXPERTS - 1


def _to_key(bits):
    return bits ^ ((bits >> 31) & jnp.int32(0x7FFFFFFF))


def _gate_block(x_ref, w_ref, b_ref, wout_ref, iout_ref):
    x = x_ref[...]
    w = w_ref[...]
    scores = jnp.dot(x, w, preferred_element_type=jnp.float32) + b_ref[...]
    bits = lax.bitcast_convert_type(scores, jnp.int32)
    eiota = lax.broadcasted_iota(jnp.int32, scores.shape, 1)
    key = (_to_key(bits) & jnp.int32(~_IDX_MASK)) | (jnp.int32(_IDX_MASK) - eiota)
    picks = []
    for _ in range(TOP_K):
        m = jnp.max(key, axis=-1, keepdims=True)
        key = jnp.where(key == m, jnp.iinfo(jnp.int32).min, key)
        picks.append(m)
    mk = jnp.concatenate(picks, axis=-1)
    iout_ref[...] = jnp.int32(_IDX_MASK) - (mk & jnp.int32(_IDX_MASK))
    sbits = _to_key(mk & jnp.int32(~_IDX_MASK))
    svals = lax.bitcast_convert_type(sbits, jnp.float32)
    e = jnp.exp(svals - svals[:, :1])
    wout_ref[...] = e / (jnp.sum(e, axis=-1, keepdims=True) + 1e-8)


@functools.partial(jax.jit, static_argnames=())
def kernel(x, gate_weight, adaptive_bias):
    orig_shape = x.shape
    xf = x.reshape(-1, orig_shape[-1])
    t = xf.shape[0]
    bt = min(BLOCK_T, t)
    wt = gate_weight.T
    bias = adaptive_bias.reshape(1, N_EXPERTS)
    grid = (pl.cdiv(t, bt),)
    wts, idx = pl.pallas_call(
        _gate_block,
        grid=grid,
        in_specs=[
            pl.BlockSpec((bt, DIM), lambda i: (i, 0)),
            pl.BlockSpec((DIM, N_EXPERTS), lambda i: (0, 0)),
            pl.BlockSpec((1, N_EXPERTS), lambda i: (0, 0)),
        ],
        out_specs=[
            pl.BlockSpec((bt, TOP_K), lambda i: (i, 0)),
            pl.BlockSpec((bt, TOP_K), lambda i: (i, 0)),
        ],
        out_shape=[
            jax.ShapeDtypeStruct((t, TOP_K), jnp.float32),
            jax.ShapeDtypeStruct((t, TOP_K), jnp.int32),
        ],
    )(xf, wt, bias)
    if len(orig_shape) == 3:
        wts = wts.reshape(orig_shape[0], orig_shape[1], TOP_K)
        idx = idx.reshape(orig_shape[0], orig_shape[1], TOP_K)
    return (wts, idx)


# speedup vs baseline: 1.5900x; 1.0341x over previous
import functools

import jax
import jax.numpy as jnp
from jax import lax
from jax.experimental import pallas as pl

DIM = 768
N_EXPERTS = 64
TOP_K = 8
BLOCK_T = 2048

_IDX_MASK = N_EXPERTS - 1


def _gate_block(x_ref, w_ref, b_ref, wout_ref, iout_ref):
    x = x_ref[...]
    w = w_ref[...]
    scores = jnp.dot(x, w, preferred_element_type=jnp.float32) + b_ref[...]
    niota = -lax.broadcasted_iota(jnp.int32, scores.shape, 1).astype(jnp.float32)
    vals = []
    negidx = []
    for _ in range(TOP_K):
        m = jnp.max(scores, axis=-1, keepdims=True)
        cand = jnp.where(scores == m, niota, -jnp.inf)
        a = jnp.max(cand, axis=-1, keepdims=True)
        scores = jnp.where(cand == a, -jnp.inf, scores)
        vals.append(m)
        negidx.append(a)
    svals = jnp.concatenate(vals, axis=-1)
    iout_ref[...] = (-jnp.concatenate(negidx, axis=-1)).astype(jnp.int32)
    e = jnp.exp(svals - svals[:, :1])
    wout_ref[...] = e / (jnp.sum(e, axis=-1, keepdims=True) + 1e-8)


@functools.partial(jax.jit, static_argnames=())
def kernel(x, gate_weight, adaptive_bias):
    orig_shape = x.shape
    xf = x.reshape(-1, orig_shape[-1])
    t = xf.shape[0]
    bt = min(BLOCK_T, t)
    wt = gate_weight.T
    bias = adaptive_bias.reshape(1, N_EXPERTS)
    grid = (pl.cdiv(t, bt),)
    wts, idx = pl.pallas_call(
        _gate_block,
        grid=grid,
        in_specs=[
            pl.BlockSpec((bt, DIM), lambda i: (i, 0)),
            pl.BlockSpec((DIM, N_EXPERTS), lambda i: (0, 0)),
            pl.BlockSpec((1, N_EXPERTS), lambda i: (0, 0)),
        ],
        out_specs=[
            pl.BlockSpec((bt, TOP_K), lambda i: (i, 0)),
            pl.BlockSpec((bt, TOP_K), lambda i: (i, 0)),
        ],
        out_shape=[
            jax.ShapeDtypeStruct((t, TOP_K), jnp.float32),
            jax.ShapeDtypeStruct((t, TOP_K), jnp.int32),
        ],
    )(xf, wt, bias)
    if len(orig_shape) == 3:
        wts = wts.reshape(orig_shape[0], orig_shape[1], TOP_K)
        idx = idx.reshape(orig_shape[0], orig_shape[1], TOP_K)
    return (wts, idx)
